# baseline (speedup 1.0000x reference)
import functools

import jax
import jax.numpy as jnp
from jax import lax
from jax.experimental import pallas as pl
from jax.experimental.pallas import tpu as pltpu
from jax.experimental.pallas import tpu_sc as plsc

SEG = 16
EMB = 1024
LANES = 16
B_TOT = 4 * 8192
NC, NS = 2, 16
NW = NC * NS
BPW = B_TOT // NW
CH = 32
NCHUNK = BPW // CH

_SCALE = float(EMB) ** 0.5

_mesh = plsc.VectorSubcoreMesh(core_axis_name="c", subcore_axis_name="s")


@functools.partial(
    pl.kernel,
    out_type=jax.ShapeDtypeStruct((B_TOT, EMB), jnp.float32),
    mesh=_mesh,
    scratch_types=[
        pltpu.VMEM((BPW + LANES,), jnp.int32),
        pltpu.VMEM((SEG, EMB), jnp.float32),
        pltpu.VMEM((CH, EMB), jnp.float32),
        pltpu.VMEM((CH, EMB), jnp.float32),
        pltpu.VMEM((CH, EMB), jnp.float32),
        pltpu.SemaphoreType.DMA,
        pltpu.SemaphoreType.DMA,
        pltpu.SemaphoreType.DMA,
    ],
)
def _emb_kernel(
    ids_hbm, w_hbm, out_hbm, idx_v, table_v, buf0, buf1, buf2, ws0, ws1, ws2
):
    wid = lax.axis_index("s") * NC + lax.axis_index("c")
    base = wid * BPW

    pltpu.sync_copy(ids_hbm.at[pl.ds(base, BPW)], idx_v.at[pl.ds(0, BPW)])
    pltpu.sync_copy(w_hbm, table_v)

    def scale_row(r, carry):
        for j in range(EMB // LANES):
            table_v[r, pl.ds(j * LANES, LANES)] = (
                table_v[r, pl.ds(j * LANES, LANES)] * _SCALE
            )
        return carry

    lax.fori_loop(0, SEG, scale_row, 0)

    def build(k, buf):
        @plsc.parallel_loop(0, CH, 1, unroll=2)
        def _row(r):
            idv = idx_v[pl.ds(k * CH + r, LANES)][0]
            for g in range(4):
                vals = [
                    table_v[idv, pl.ds((g * 16 + j) * LANES, LANES)]
                    for j in range(16)
                ]
                for j in range(16):
                    buf[r, pl.ds((g * 16 + j) * LANES, LANES)] = vals[j]

    def w_start(k, buf, sem):
        pltpu.async_copy(buf, out_hbm.at[pl.ds(base + k * CH, CH)], sem)

    def w_wait(k, buf, sem):
        pltpu.make_async_copy(buf, out_hbm.at[pl.ds(base + k * CH, CH)], sem).wait()

    bufs = (buf0, buf1)
    sems = (ws0, ws1)
    NBUF = len(bufs)

    def do_chunk(k, buf, sem):
        build(k, buf)

    def step(k, carry):
        for b in range(NBUF):
            @pl.when(k % NBUF == b)
            def _(b=b):
                do_chunk(k, bufs[b], sems[b])

        return carry

    lax.fori_loop(0, NCHUNK, step, 0)

    w_start(0, buf0, ws0)
    w_wait(0, buf0, ws0)


def kernel(segment_ids, weight):
    ids_flat = segment_ids.reshape(-1).astype(jnp.int32)
    out = _emb_kernel(ids_flat, weight)
    return out.reshape(segment_ids.shape + (EMB,))


# speedup vs baseline: 1.4447x; 1.4447x over previous
import functools

import jax
import jax.numpy as jnp
from jax import lax
from jax.experimental import pallas as pl
from jax.experimental.pallas import tpu as pltpu
from jax.experimental.pallas import tpu_sc as plsc

SEG = 16
EMB = 1024
LANES = 16
B_TOT = 4 * 8192
NC, NS = 2, 16
NW = NC * NS
BPW = B_TOT // NW

_SCALE = float(EMB) ** 0.5

_mesh = plsc.VectorSubcoreMesh(core_axis_name="c", subcore_axis_name="s")


@functools.partial(
    pl.kernel,
    out_type=jax.ShapeDtypeStruct((B_TOT, EMB), jnp.float32),
    mesh=_mesh,
    scratch_types=[
        pltpu.VMEM((BPW + LANES,), jnp.int32),
        pltpu.VMEM((SEG, EMB), jnp.float32),
        pltpu.SemaphoreType.DMA,
    ],
)
def _emb_kernel(ids_hbm, w_hbm, out_hbm, idx_v, table_v, sem):
    wid = lax.axis_index("s") * NC + lax.axis_index("c")
    base = wid * BPW

    pltpu.sync_copy(ids_hbm.at[pl.ds(base, BPW)], idx_v.at[pl.ds(0, BPW)])
    pltpu.sync_copy(w_hbm, table_v)

    def scale_row(r, carry):
        for j in range(EMB // LANES):
            table_v[r, pl.ds(j * LANES, LANES)] = (
                table_v[r, pl.ds(j * LANES, LANES)] * _SCALE
            )
        return carry

    lax.fori_loop(0, SEG, scale_row, 0)

    def row(r, carry):
        idv = idx_v[pl.ds(r, LANES)][0]
        pltpu.async_copy(table_v.at[idv], out_hbm.at[base + r], sem)
        return carry

    lax.fori_loop(0, BPW, row, 0)

    def drain(r, carry):
        pltpu.make_async_copy(table_v.at[0], out_hbm.at[base + r], sem).wait()
        return carry

    lax.fori_loop(0, BPW, drain, 0)


def kernel(segment_ids, weight):
    ids_flat = segment_ids.reshape(-1).astype(jnp.int32)
    out = _emb_kernel(ids_flat, weight)
    return out.reshape(segment_ids.shape + (EMB,))
